# Initial kernel scaffold; baseline (speedup 1.0000x reference)
import jax
import jax.numpy as jnp
from jax.experimental import pallas as pl


def kernel(w1, b1, w2, b2, w3, b3, bn1_g, bn1_b, bn2_g, bn2_b, bnsc_g, bnsc_b, x):
    raise NotImplementedError("write your pallas kernel here")


# speedup vs baseline: 2.2210x; 2.2210x over previous
import functools

import jax
import jax.numpy as jnp
from jax.experimental import pallas as pl
from jax.experimental.pallas import tpu as pltpu

F32 = jnp.float32
BF16 = jnp.bfloat16
EPS = 1e-5
MARGIN = 128


def _round_up(a, b):
    return (a + b - 1) // b * b


def _stats_body(x_ref, ssum_ref, ssq_ref):
    x = x_ref[0]
    ssum_ref[0] = jnp.sum(x, axis=1, keepdims=True)
    ssq_ref[0] = jnp.sum(x * x, axis=1, keepdims=True)


def _col_masks(w, hw):
    c = jax.lax.broadcasted_iota(jnp.int32, (1, hw), 1) % w
    return (c >= 1).astype(BF16), (c <= w - 2).astype(BF16)


def _fill_taps(col_ref, ybuf_ref, y, base, w, hw, ch, mask_l, mask_r):
    ybuf_ref[...] = jnp.zeros_like(ybuf_ref)
    ybuf_ref[:, MARGIN:MARGIN + hw] = y
    for kh in range(3):
        for kw in range(3):
            t = kh * 3 + kw
            off = (kh - 1) * w + (kw - 1)
            win = ybuf_ref[:, MARGIN + off:MARGIN + off + hw]
            if kw == 0:
                win = win * mask_l
            elif kw == 2:
                win = win * mask_r
            col_ref[base + t * ch:base + (t + 1) * ch, :] = win


def _conv1_body(x_ref, sc_ref, sh_ref, w_ref, b_ref,
                h1_ref, ssum_ref, ssq_ref, ybuf_ref, col_ref, *, w, hw, cin):
    mask_l, mask_r = _col_masks(w, hw)
    y = jnp.maximum(x_ref[0] * sc_ref[...] + sh_ref[...], 0.0).astype(BF16)
    _fill_taps(col_ref, ybuf_ref, y, 0, w, hw, cin, mask_l, mask_r)
    acc = jnp.dot(w_ref[...], col_ref[...], preferred_element_type=F32)
    acc = acc + b_ref[...]
    ssum_ref[0] = jnp.sum(acc, axis=1, keepdims=True)
    ssq_ref[0] = jnp.sum(acc * acc, axis=1, keepdims=True)
    h1_ref[0] = acc.astype(BF16)


def _conv23_body(h1_ref, x_ref, sc2_ref, sh2_ref, scs_ref, shs_ref,
                 w_ref, b_ref, out_ref, ybuf_ref, col_ref, *, w, hw, cin, cout):
    mask_l, mask_r = _col_masks(w, hw)
    y2 = jnp.maximum(h1_ref[0].astype(F32) * sc2_ref[...] + sh2_ref[...],
                     0.0).astype(BF16)
    _fill_taps(col_ref, ybuf_ref, y2, 0, w, hw, cout, mask_l, mask_r)
    y3 = jnp.maximum(x_ref[0] * scs_ref[...] + shs_ref[...], 0.0).astype(BF16)
    _fill_taps(col_ref, ybuf_ref, y3, 9 * cout, w, hw, cin, mask_l, mask_r)
    acc = jnp.dot(w_ref[...], col_ref[...], preferred_element_type=F32)
    out_ref[0] = acc + b_ref[...]


def _wmat(wt):
    co, ci, kh, kw = wt.shape
    return jnp.transpose(wt, (0, 2, 3, 1)).reshape(co, kh * kw * ci)


def _scale_shift(ssum, ssq, count, gamma, beta):
    s = jnp.sum(ssum[:, :, 0], axis=0)
    q = jnp.sum(ssq[:, :, 0], axis=0)
    mean = s / count
    var = jnp.maximum(q / count - mean * mean, 0.0)
    scale = gamma * jax.lax.rsqrt(var + EPS)
    shift = beta - mean * scale
    return scale[:, None], shift[:, None]


def kernel(w1, b1, w2, b2, w3, b3, bn1_g, bn1_b, bn2_g, bn2_b,
           bnsc_g, bnsc_b, x):
    n, cin, h, w = x.shape
    cout = w1.shape[0]
    hw = h * w
    lbuf = _round_up(MARGIN + hw + w + 2, 128)
    xr = x.reshape(n, cin, hw)

    sx, qx = pl.pallas_call(
        _stats_body,
        out_shape=(jax.ShapeDtypeStruct((n, cin, 1), F32),
                   jax.ShapeDtypeStruct((n, cin, 1), F32)),
        grid=(n,),
        in_specs=[pl.BlockSpec((1, cin, hw), lambda i: (i, 0, 0))],
        out_specs=(pl.BlockSpec((1, cin, 1), lambda i: (i, 0, 0)),
                   pl.BlockSpec((1, cin, 1), lambda i: (i, 0, 0))),
        compiler_params=pltpu.CompilerParams(dimension_semantics=("parallel",)),
    )(xr)
    sc1, sh1 = _scale_shift(sx, qx, n * hw, bn1_g, bn1_b)
    scs, shs = _scale_shift(sx, qx, n * hw, bnsc_g, bnsc_b)

    w1m = _wmat(w1).astype(BF16)
    h1, s1, q1 = pl.pallas_call(
        functools.partial(_conv1_body, w=w, hw=hw, cin=cin),
        out_shape=(jax.ShapeDtypeStruct((n, cout, hw), BF16),
                   jax.ShapeDtypeStruct((n, cout, 1), F32),
                   jax.ShapeDtypeStruct((n, cout, 1), F32)),
        grid=(n,),
        in_specs=[pl.BlockSpec((1, cin, hw), lambda i: (i, 0, 0)),
                  pl.BlockSpec((cin, 1), lambda i: (0, 0)),
                  pl.BlockSpec((cin, 1), lambda i: (0, 0)),
                  pl.BlockSpec((cout, 9 * cin), lambda i: (0, 0)),
                  pl.BlockSpec((cout, 1), lambda i: (0, 0))],
        out_specs=(pl.BlockSpec((1, cout, hw), lambda i: (i, 0, 0)),
                   pl.BlockSpec((1, cout, 1), lambda i: (i, 0, 0)),
                   pl.BlockSpec((1, cout, 1), lambda i: (i, 0, 0))),
        scratch_shapes=[pltpu.VMEM((cin, lbuf), BF16),
                        pltpu.VMEM((9 * cin, hw), BF16)],
        compiler_params=pltpu.CompilerParams(
            dimension_semantics=("parallel",),
            vmem_limit_bytes=64 * 1024 * 1024),
    )(xr, sc1, sh1, w1m, b1[:, None])
    sc2, sh2 = _scale_shift(s1, q1, n * hw, bn2_g, bn2_b)

    w23 = jnp.concatenate([_wmat(w2), _wmat(w3)], axis=1).astype(BF16)
    b23 = (b2 + b3)[:, None]
    out = pl.pallas_call(
        functools.partial(_conv23_body, w=w, hw=hw, cin=cin, cout=cout),
        out_shape=jax.ShapeDtypeStruct((n, cout, hw), F32),
        grid=(n,),
        in_specs=[pl.BlockSpec((1, cout, hw), lambda i: (i, 0, 0)),
                  pl.BlockSpec((1, cin, hw), lambda i: (i, 0, 0)),
                  pl.BlockSpec((cout, 1), lambda i: (0, 0)),
                  pl.BlockSpec((cout, 1), lambda i: (0, 0)),
                  pl.BlockSpec((cin, 1), lambda i: (0, 0)),
                  pl.BlockSpec((cin, 1), lambda i: (0, 0)),
                  pl.BlockSpec((cout, 9 * (cin + cout)), lambda i: (0, 0)),
                  pl.BlockSpec((cout, 1), lambda i: (0, 0))],
        out_specs=pl.BlockSpec((1, cout, hw), lambda i: (i, 0, 0)),
        scratch_shapes=[pltpu.VMEM((max(cin, cout), lbuf), BF16),
                        pltpu.VMEM((9 * (cin + cout), hw), BF16)],
        compiler_params=pltpu.CompilerParams(
            dimension_semantics=("parallel",),
            vmem_limit_bytes=64 * 1024 * 1024),
    )(h1, xr, sc2, sh2, scs, shs, w23, b23)

    return out.reshape(n, cout, h, w)


# speedup vs baseline: 2.5425x; 1.1447x over previous
import functools

import jax
import jax.numpy as jnp
from jax.experimental import pallas as pl
from jax.experimental.pallas import tpu as pltpu

F32 = jnp.float32
BF16 = jnp.bfloat16
EPS = 1e-5
MARGIN = 128


def _round_up(a, b):
    return (a + b - 1) // b * b


def _stats_body(x_ref, ssum_ref, ssq_ref):
    x = x_ref[0]
    ssum_ref[0] = jnp.sum(x, axis=1, keepdims=True)
    ssq_ref[0] = jnp.sum(x * x, axis=1, keepdims=True)


def _fill_buf(buf_ref, y, w, hw, ch):
    q = jax.lax.broadcasted_iota(jnp.int32, (1, hw), 1) % w
    buf_ref[:, :MARGIN + 2] = jnp.zeros_like(buf_ref[:, :MARGIN + 2])
    buf_ref[:, MARGIN + hw - 2:] = jnp.zeros_like(buf_ref[:, MARGIN + hw - 2:])
    buf_ref[0 * ch:1 * ch, MARGIN + 1:MARGIN + 1 + hw] = (
        y * (q != w - 1).astype(BF16))
    buf_ref[1 * ch:2 * ch, MARGIN:MARGIN + hw] = y
    buf_ref[2 * ch:3 * ch, MARGIN - 1:MARGIN - 1 + hw] = (
        y * (q != 0).astype(BF16))


def _row_dots(w_ref, buf_ref, base, w, hw, acc):
    for kh in range(3):
        lo = MARGIN + (kh - 1) * w
        acc = acc + jnp.dot(w_ref[base + kh], buf_ref[:, lo:lo + hw],
                            preferred_element_type=F32)
    return acc


def _conv1_body(x_ref, sc_ref, sh_ref, w_ref, b_ref,
                h1_ref, ssum_ref, ssq_ref, buf_ref, *, w, hw, cin):
    y = jnp.maximum(x_ref[0] * sc_ref[...] + sh_ref[...], 0.0).astype(BF16)
    _fill_buf(buf_ref, y, w, hw, cin)
    acc = _row_dots(w_ref, buf_ref, 0, w, hw, b_ref[...])
    ssum_ref[0] = jnp.sum(acc, axis=1, keepdims=True)
    ssq_ref[0] = jnp.sum(acc * acc, axis=1, keepdims=True)
    h1_ref[0] = acc.astype(BF16)


def _conv23_body(h1_ref, x_ref, sc2_ref, sh2_ref, scs_ref, shs_ref,
                 w_ref, b_ref, out_ref, buf2_ref, buf3_ref, *, w, hw, cin, cout):
    y2 = jnp.maximum(h1_ref[0].astype(F32) * sc2_ref[...] + sh2_ref[...],
                     0.0).astype(BF16)
    _fill_buf(buf2_ref, y2, w, hw, cout)
    acc = _row_dots(w_ref, buf2_ref, 0, w, hw, b_ref[...])
    y3 = jnp.maximum(x_ref[0] * scs_ref[...] + shs_ref[...], 0.0).astype(BF16)
    _fill_buf(buf3_ref, y3, w, hw, cin)
    acc = _row_dots(w_ref, buf3_ref, 3, w, hw, acc)
    out_ref[0] = acc


def _wrows(wt):
    co, ci, kh, kw = wt.shape
    return jnp.transpose(wt, (2, 0, 3, 1)).reshape(kh, co, kw * ci)


def _scale_shift(ssum, ssq, count, gamma, beta):
    s = jnp.sum(ssum[:, :, 0], axis=0)
    q = jnp.sum(ssq[:, :, 0], axis=0)
    mean = s / count
    var = jnp.maximum(q / count - mean * mean, 0.0)
    scale = gamma * jax.lax.rsqrt(var + EPS)
    shift = beta - mean * scale
    return scale[:, None], shift[:, None]


def kernel(w1, b1, w2, b2, w3, b3, bn1_g, bn1_b, bn2_g, bn2_b,
           bnsc_g, bnsc_b, x):
    n, cin, h, w = x.shape
    cout = w1.shape[0]
    hw = h * w
    lbuf = _round_up(MARGIN + hw + w + 2, 128)
    xr = x.reshape(n, cin, hw)

    sx, qx = pl.pallas_call(
        _stats_body,
        out_shape=(jax.ShapeDtypeStruct((n, cin, 1), F32),
                   jax.ShapeDtypeStruct((n, cin, 1), F32)),
        grid=(n,),
        in_specs=[pl.BlockSpec((1, cin, hw), lambda i: (i, 0, 0))],
        out_specs=(pl.BlockSpec((1, cin, 1), lambda i: (i, 0, 0)),
                   pl.BlockSpec((1, cin, 1), lambda i: (i, 0, 0))),
        compiler_params=pltpu.CompilerParams(dimension_semantics=("parallel",)),
    )(xr)
    sc1, sh1 = _scale_shift(sx, qx, n * hw, bn1_g, bn1_b)
    scs, shs = _scale_shift(sx, qx, n * hw, bnsc_g, bnsc_b)

    w1r = _wrows(w1).astype(BF16)
    h1, s1, q1 = pl.pallas_call(
        functools.partial(_conv1_body, w=w, hw=hw, cin=cin),
        out_shape=(jax.ShapeDtypeStruct((n, cout, hw), BF16),
                   jax.ShapeDtypeStruct((n, cout, 1), F32),
                   jax.ShapeDtypeStruct((n, cout, 1), F32)),
        grid=(n,),
        in_specs=[pl.BlockSpec((1, cin, hw), lambda i: (i, 0, 0)),
                  pl.BlockSpec((cin, 1), lambda i: (0, 0)),
                  pl.BlockSpec((cin, 1), lambda i: (0, 0)),
                  pl.BlockSpec((3, cout, 3 * cin), lambda i: (0, 0, 0)),
                  pl.BlockSpec((cout, 1), lambda i: (0, 0))],
        out_specs=(pl.BlockSpec((1, cout, hw), lambda i: (i, 0, 0)),
                   pl.BlockSpec((1, cout, 1), lambda i: (i, 0, 0)),
                   pl.BlockSpec((1, cout, 1), lambda i: (i, 0, 0))),
        scratch_shapes=[pltpu.VMEM((3 * cin, lbuf), BF16)],
        compiler_params=pltpu.CompilerParams(
            dimension_semantics=("parallel",),
            vmem_limit_bytes=64 * 1024 * 1024),
    )(xr, sc1, sh1, w1r, b1[:, None])
    sc2, sh2 = _scale_shift(s1, q1, n * hw, bn2_g, bn2_b)

    w23r = jnp.concatenate([_wrows(w2), _wrows(w3)], axis=0).astype(BF16)
    b23 = (b2 + b3)[:, None]
    out = pl.pallas_call(
        functools.partial(_conv23_body, w=w, hw=hw, cin=cin, cout=cout),
        out_shape=jax.ShapeDtypeStruct((n, cout, hw), F32),
        grid=(n,),
        in_specs=[pl.BlockSpec((1, cout, hw), lambda i: (i, 0, 0)),
                  pl.BlockSpec((1, cin, hw), lambda i: (i, 0, 0)),
                  pl.BlockSpec((cout, 1), lambda i: (0, 0)),
                  pl.BlockSpec((cout, 1), lambda i: (0, 0)),
                  pl.BlockSpec((cin, 1), lambda i: (0, 0)),
                  pl.BlockSpec((cin, 1), lambda i: (0, 0)),
                  pl.BlockSpec((6, cout, 3 * cin), lambda i: (0, 0, 0)),
                  pl.BlockSpec((cout, 1), lambda i: (0, 0))],
        out_specs=pl.BlockSpec((1, cout, hw), lambda i: (i, 0, 0)),
        scratch_shapes=[pltpu.VMEM((3 * cout, lbuf), BF16),
                        pltpu.VMEM((3 * cin, lbuf), BF16)],
        compiler_params=pltpu.CompilerParams(
            dimension_semantics=("parallel",),
            vmem_limit_bytes=64 * 1024 * 1024),
    )(h1, xr, sc2, sh2, scs, shs, w23r, b23)

    return out.reshape(n, cout, h, w)


# speedup vs baseline: 3.0857x; 1.2136x over previous
import functools

import jax
import jax.numpy as jnp
from jax.experimental import pallas as pl
from jax.experimental.pallas import tpu as pltpu

F32 = jnp.float32
BF16 = jnp.bfloat16
EPS = 1e-5
MARGIN = 128


def _round_up(a, b):
    return (a + b - 1) // b * b


def _stats_body(x_ref, ssum_ref, ssq_ref):
    x = x_ref[0]
    ssum_ref[0] = jnp.sum(x, axis=1, keepdims=True)
    ssq_ref[0] = jnp.sum(x * x, axis=1, keepdims=True)


def _fill_buf(buf_ref, y, row, w, hw, ch):
    q = jax.lax.broadcasted_iota(jnp.int32, (1, hw), 1) % w
    sl = buf_ref[row:row + 3 * ch, :MARGIN + 2]
    buf_ref[row:row + 3 * ch, :MARGIN + 2] = jnp.zeros_like(sl)
    sr = buf_ref[row:row + 3 * ch, MARGIN + hw - 2:]
    buf_ref[row:row + 3 * ch, MARGIN + hw - 2:] = jnp.zeros_like(sr)
    buf_ref[row + 0 * ch:row + 1 * ch, MARGIN + 1:MARGIN + 1 + hw] = (
        y * (q != w - 1).astype(BF16))
    buf_ref[row + 1 * ch:row + 2 * ch, MARGIN:MARGIN + hw] = y
    buf_ref[row + 2 * ch:row + 3 * ch, MARGIN - 1:MARGIN - 1 + hw] = (
        y * (q != 0).astype(BF16))


def _combine_rows(zbuf_ref, bias, w, hw, cout):
    acc = bias + zbuf_ref[cout:2 * cout, MARGIN:MARGIN + hw]
    acc = acc + zbuf_ref[0:cout, MARGIN - w:MARGIN - w + hw]
    acc = acc + zbuf_ref[2 * cout:3 * cout, MARGIN + w:MARGIN + w + hw]
    return acc


def _zstore(zbuf_ref, z, hw):
    zl = zbuf_ref[:, :MARGIN]
    zbuf_ref[:, :MARGIN] = jnp.zeros_like(zl)
    zr = zbuf_ref[:, MARGIN + hw:]
    zbuf_ref[:, MARGIN + hw:] = jnp.zeros_like(zr)
    zbuf_ref[:, MARGIN:MARGIN + hw] = z


def _conv1_body(x_ref, sc_ref, sh_ref, w_ref, b_ref,
                h1_ref, ssum_ref, ssq_ref, buf_ref, zbuf_ref, *, w, hw, cin,
                cout):
    y = jnp.maximum(x_ref[0] * sc_ref[...] + sh_ref[...], 0.0).astype(BF16)
    _fill_buf(buf_ref, y, 0, w, hw, cin)
    z = jnp.dot(w_ref[...], buf_ref[:, MARGIN:MARGIN + hw],
                preferred_element_type=F32)
    _zstore(zbuf_ref, z, hw)
    acc = _combine_rows(zbuf_ref, b_ref[...], w, hw, cout)
    ssum_ref[0] = jnp.sum(acc, axis=1, keepdims=True)
    ssq_ref[0] = jnp.sum(acc * acc, axis=1, keepdims=True)
    h1_ref[0] = acc.astype(BF16)


def _conv23_body(h1_ref, x_ref, sc2_ref, sh2_ref, scs_ref, shs_ref,
                 w_ref, b_ref, out_ref, buf_ref, zbuf_ref, *, w, hw, cin, cout):
    y2 = jnp.maximum(h1_ref[0].astype(F32) * sc2_ref[...] + sh2_ref[...],
                     0.0).astype(BF16)
    _fill_buf(buf_ref, y2, 0, w, hw, cout)
    y3 = jnp.maximum(x_ref[0] * scs_ref[...] + shs_ref[...], 0.0).astype(BF16)
    _fill_buf(buf_ref, y3, 3 * cout, w, hw, cin)
    z = jnp.dot(w_ref[...], buf_ref[:, MARGIN:MARGIN + hw],
                preferred_element_type=F32)
    _zstore(zbuf_ref, z, hw)
    out_ref[0] = _combine_rows(zbuf_ref, b_ref[...], w, hw, cout)


def _wcat(wt):
    co, ci, kh, kw = wt.shape
    return jnp.transpose(wt, (2, 0, 3, 1)).reshape(kh * co, kw * ci)


def _scale_shift(ssum, ssq, count, gamma, beta):
    s = jnp.sum(ssum[:, :, 0], axis=0)
    q = jnp.sum(ssq[:, :, 0], axis=0)
    mean = s / count
    var = jnp.maximum(q / count - mean * mean, 0.0)
    scale = gamma * jax.lax.rsqrt(var + EPS)
    shift = beta - mean * scale
    return scale[:, None], shift[:, None]


def kernel(w1, b1, w2, b2, w3, b3, bn1_g, bn1_b, bn2_g, bn2_b,
           bnsc_g, bnsc_b, x):
    n, cin, h, w = x.shape
    cout = w1.shape[0]
    hw = h * w
    lbuf = _round_up(MARGIN + hw + w + 2, 128)
    xr = x.reshape(n, cin, hw)

    sx, qx = pl.pallas_call(
        _stats_body,
        out_shape=(jax.ShapeDtypeStruct((n, cin, 1), F32),
                   jax.ShapeDtypeStruct((n, cin, 1), F32)),
        grid=(n,),
        in_specs=[pl.BlockSpec((1, cin, hw), lambda i: (i, 0, 0))],
        out_specs=(pl.BlockSpec((1, cin, 1), lambda i: (i, 0, 0)),
                   pl.BlockSpec((1, cin, 1), lambda i: (i, 0, 0))),
        compiler_params=pltpu.CompilerParams(dimension_semantics=("parallel",)),
    )(xr)
    sc1, sh1 = _scale_shift(sx, qx, n * hw, bn1_g, bn1_b)
    scs, shs = _scale_shift(sx, qx, n * hw, bnsc_g, bnsc_b)

    w1c = _wcat(w1).astype(BF16)
    h1, s1, q1 = pl.pallas_call(
        functools.partial(_conv1_body, w=w, hw=hw, cin=cin, cout=cout),
        out_shape=(jax.ShapeDtypeStruct((n, cout, hw), BF16),
                   jax.ShapeDtypeStruct((n, cout, 1), F32),
                   jax.ShapeDtypeStruct((n, cout, 1), F32)),
        grid=(n,),
        in_specs=[pl.BlockSpec((1, cin, hw), lambda i: (i, 0, 0)),
                  pl.BlockSpec((cin, 1), lambda i: (0, 0)),
                  pl.BlockSpec((cin, 1), lambda i: (0, 0)),
                  pl.BlockSpec((3 * cout, 3 * cin), lambda i: (0, 0)),
                  pl.BlockSpec((cout, 1), lambda i: (0, 0))],
        out_specs=(pl.BlockSpec((1, cout, hw), lambda i: (i, 0, 0)),
                   pl.BlockSpec((1, cout, 1), lambda i: (i, 0, 0)),
                   pl.BlockSpec((1, cout, 1), lambda i: (i, 0, 0))),
        scratch_shapes=[pltpu.VMEM((3 * cin, lbuf), BF16),
                        pltpu.VMEM((3 * cout, lbuf), F32)],
        compiler_params=pltpu.CompilerParams(
            dimension_semantics=("parallel",),
            vmem_limit_bytes=64 * 1024 * 1024),
    )(xr, sc1, sh1, w1c, b1[:, None])
    sc2, sh2 = _scale_shift(s1, q1, n * hw, bn2_g, bn2_b)

    w23c = jnp.concatenate([_wcat(w2), _wcat(w3)], axis=1).astype(BF16)
    b23 = (b2 + b3)[:, None]
    out = pl.pallas_call(
        functools.partial(_conv23_body, w=w, hw=hw, cin=cin, cout=cout),
        out_shape=jax.ShapeDtypeStruct((n, cout, hw), F32),
        grid=(n,),
        in_specs=[pl.BlockSpec((1, cout, hw), lambda i: (i, 0, 0)),
                  pl.BlockSpec((1, cin, hw), lambda i: (i, 0, 0)),
                  pl.BlockSpec((cout, 1), lambda i: (0, 0)),
                  pl.BlockSpec((cout, 1), lambda i: (0, 0)),
                  pl.BlockSpec((cin, 1), lambda i: (0, 0)),
                  pl.BlockSpec((cin, 1), lambda i: (0, 0)),
                  pl.BlockSpec((3 * cout, 3 * (cin + cout)), lambda i: (0, 0)),
                  pl.BlockSpec((cout, 1), lambda i: (0, 0))],
        out_specs=pl.BlockSpec((1, cout, hw), lambda i: (i, 0, 0)),
        scratch_shapes=[pltpu.VMEM((3 * (cin + cout), lbuf), BF16),
                        pltpu.VMEM((3 * cout, lbuf), F32)],
        compiler_params=pltpu.CompilerParams(
            dimension_semantics=("parallel",),
            vmem_limit_bytes=64 * 1024 * 1024),
    )(h1, xr, sc2, sh2, scs, shs, w23c, b23)

    return out.reshape(n, cout, h, w)


# speedup vs baseline: 3.0858x; 1.0000x over previous
import functools

import jax
import jax.numpy as jnp
from jax.experimental import pallas as pl
from jax.experimental.pallas import tpu as pltpu

F32 = jnp.float32
BF16 = jnp.bfloat16
EPS = 1e-5
MARGIN = 128


def _round_up(a, b):
    return (a + b - 1) // b * b


def _stats_body(x_ref, ssum_ref, ssq_ref, xbf_ref):
    x = x_ref[0]
    ssum_ref[0] = jnp.sum(x, axis=1, keepdims=True)
    ssq_ref[0] = jnp.sum(x * x, axis=1, keepdims=True)
    xbf_ref[0] = x.astype(BF16)


def _fill_buf(buf_ref, y, row, w, hw, ch):
    q = jax.lax.broadcasted_iota(jnp.int32, (1, hw), 1) % w
    sl = buf_ref[row:row + 3 * ch, :MARGIN + 2]
    buf_ref[row:row + 3 * ch, :MARGIN + 2] = jnp.zeros_like(sl)
    sr = buf_ref[row:row + 3 * ch, MARGIN + hw - 2:]
    buf_ref[row:row + 3 * ch, MARGIN + hw - 2:] = jnp.zeros_like(sr)
    buf_ref[row + 0 * ch:row + 1 * ch, MARGIN + 1:MARGIN + 1 + hw] = (
        y * (q != w - 1).astype(BF16))
    buf_ref[row + 1 * ch:row + 2 * ch, MARGIN:MARGIN + hw] = y
    buf_ref[row + 2 * ch:row + 3 * ch, MARGIN - 1:MARGIN - 1 + hw] = (
        y * (q != 0).astype(BF16))


def _combine_rows(zbuf_ref, bias, w, hw, cout):
    acc = bias + zbuf_ref[cout:2 * cout, MARGIN:MARGIN + hw]
    acc = acc + zbuf_ref[0:cout, MARGIN - w:MARGIN - w + hw]
    acc = acc + zbuf_ref[2 * cout:3 * cout, MARGIN + w:MARGIN + w + hw]
    return acc


def _zstore(zbuf_ref, z, hw):
    zl = zbuf_ref[:, :MARGIN]
    zbuf_ref[:, :MARGIN] = jnp.zeros_like(zl)
    zr = zbuf_ref[:, MARGIN + hw:]
    zbuf_ref[:, MARGIN + hw:] = jnp.zeros_like(zr)
    zbuf_ref[:, MARGIN:MARGIN + hw] = z


def _conv1_body(x_ref, sc_ref, sh_ref, w_ref, b_ref,
                h1_ref, ssum_ref, ssq_ref, buf_ref, zbuf_ref, *, w, hw, cin,
                cout):
    y = jnp.maximum(x_ref[0].astype(F32) * sc_ref[...] + sh_ref[...],
                    0.0).astype(BF16)
    _fill_buf(buf_ref, y, 0, w, hw, cin)
    z = jnp.dot(w_ref[...], buf_ref[:, MARGIN:MARGIN + hw],
                preferred_element_type=F32)
    _zstore(zbuf_ref, z, hw)
    acc = _combine_rows(zbuf_ref, b_ref[...], w, hw, cout)
    ssum_ref[0] = jnp.sum(acc, axis=1, keepdims=True)
    ssq_ref[0] = jnp.sum(acc * acc, axis=1, keepdims=True)
    h1_ref[0] = acc.astype(BF16)


def _conv23_body(h1_ref, x_ref, sc2_ref, sh2_ref, scs_ref, shs_ref,
                 w_ref, b_ref, out_ref, buf_ref, zbuf_ref, *, w, hw, cin, cout):
    y2 = jnp.maximum(h1_ref[0].astype(F32) * sc2_ref[...] + sh2_ref[...],
                     0.0).astype(BF16)
    _fill_buf(buf_ref, y2, 0, w, hw, cout)
    y3 = jnp.maximum(x_ref[0].astype(F32) * scs_ref[...] + shs_ref[...],
                     0.0).astype(BF16)
    _fill_buf(buf_ref, y3, 3 * cout, w, hw, cin)
    z = jnp.dot(w_ref[...], buf_ref[:, MARGIN:MARGIN + hw],
                preferred_element_type=F32)
    _zstore(zbuf_ref, z, hw)
    out_ref[0] = _combine_rows(zbuf_ref, b_ref[...], w, hw, cout)


def _wcat(wt):
    co, ci, kh, kw = wt.shape
    return jnp.transpose(wt, (2, 0, 3, 1)).reshape(kh * co, kw * ci)


def _scale_shift(ssum, ssq, count, gamma, beta):
    s = jnp.sum(ssum[:, :, 0], axis=0)
    q = jnp.sum(ssq[:, :, 0], axis=0)
    mean = s / count
    var = jnp.maximum(q / count - mean * mean, 0.0)
    scale = gamma * jax.lax.rsqrt(var + EPS)
    shift = beta - mean * scale
    return scale[:, None], shift[:, None]


def kernel(w1, b1, w2, b2, w3, b3, bn1_g, bn1_b, bn2_g, bn2_b,
           bnsc_g, bnsc_b, x):
    n, cin, h, w = x.shape
    cout = w1.shape[0]
    hw = h * w
    lbuf = _round_up(MARGIN + hw + w + 2, 128)
    xr = x.reshape(n, cin, hw)

    sx, qx, xbf = pl.pallas_call(
        _stats_body,
        out_shape=(jax.ShapeDtypeStruct((n, cin, 1), F32),
                   jax.ShapeDtypeStruct((n, cin, 1), F32),
                   jax.ShapeDtypeStruct((n, cin, hw), BF16)),
        grid=(n,),
        in_specs=[pl.BlockSpec((1, cin, hw), lambda i: (i, 0, 0))],
        out_specs=(pl.BlockSpec((1, cin, 1), lambda i: (i, 0, 0)),
                   pl.BlockSpec((1, cin, 1), lambda i: (i, 0, 0)),
                   pl.BlockSpec((1, cin, hw), lambda i: (i, 0, 0))),
        compiler_params=pltpu.CompilerParams(dimension_semantics=("parallel",)),
    )(xr)
    sc1, sh1 = _scale_shift(sx, qx, n * hw, bn1_g, bn1_b)
    scs, shs = _scale_shift(sx, qx, n * hw, bnsc_g, bnsc_b)

    w1c = _wcat(w1).astype(BF16)
    h1, s1, q1 = pl.pallas_call(
        functools.partial(_conv1_body, w=w, hw=hw, cin=cin, cout=cout),
        out_shape=(jax.ShapeDtypeStruct((n, cout, hw), BF16),
                   jax.ShapeDtypeStruct((n, cout, 1), F32),
                   jax.ShapeDtypeStruct((n, cout, 1), F32)),
        grid=(n,),
        in_specs=[pl.BlockSpec((1, cin, hw), lambda i: (i, 0, 0)),
                  pl.BlockSpec((cin, 1), lambda i: (0, 0)),
                  pl.BlockSpec((cin, 1), lambda i: (0, 0)),
                  pl.BlockSpec((3 * cout, 3 * cin), lambda i: (0, 0)),
                  pl.BlockSpec((cout, 1), lambda i: (0, 0))],
        out_specs=(pl.BlockSpec((1, cout, hw), lambda i: (i, 0, 0)),
                   pl.BlockSpec((1, cout, 1), lambda i: (i, 0, 0)),
                   pl.BlockSpec((1, cout, 1), lambda i: (i, 0, 0))),
        scratch_shapes=[pltpu.VMEM((3 * cin, lbuf), BF16),
                        pltpu.VMEM((3 * cout, lbuf), F32)],
        compiler_params=pltpu.CompilerParams(
            dimension_semantics=("parallel",),
            vmem_limit_bytes=64 * 1024 * 1024),
    )(xbf, sc1, sh1, w1c, b1[:, None])
    sc2, sh2 = _scale_shift(s1, q1, n * hw, bn2_g, bn2_b)

    w23c = jnp.concatenate([_wcat(w2), _wcat(w3)], axis=1).astype(BF16)
    b23 = (b2 + b3)[:, None]
    out = pl.pallas_call(
        functools.partial(_conv23_body, w=w, hw=hw, cin=cin, cout=cout),
        out_shape=jax.ShapeDtypeStruct((n, cout, hw), F32),
        grid=(n,),
        in_specs=[pl.BlockSpec((1, cout, hw), lambda i: (i, 0, 0)),
                  pl.BlockSpec((1, cin, hw), lambda i: (i, 0, 0)),
                  pl.BlockSpec((cout, 1), lambda i: (0, 0)),
                  pl.BlockSpec((cout, 1), lambda i: (0, 0)),
                  pl.BlockSpec((cin, 1), lambda i: (0, 0)),
                  pl.BlockSpec((cin, 1), lambda i: (0, 0)),
                  pl.BlockSpec((3 * cout, 3 * (cin + cout)), lambda i: (0, 0)),
                  pl.BlockSpec((cout, 1), lambda i: (0, 0))],
        out_specs=pl.BlockSpec((1, cout, hw), lambda i: (i, 0, 0)),
        scratch_shapes=[pltpu.VMEM((3 * (cin + cout), lbuf), BF16),
                        pltpu.VMEM((3 * cout, lbuf), F32)],
        compiler_params=pltpu.CompilerParams(
            dimension_semantics=("parallel",),
            vmem_limit_bytes=64 * 1024 * 1024),
    )(h1, xbf, sc2, sh2, scs, shs, w23c, b23)

    return out.reshape(n, cout, h, w)


# speedup vs baseline: 3.1016x; 1.0051x over previous
import functools

import jax
import jax.numpy as jnp
from jax.experimental import pallas as pl
from jax.experimental.pallas import tpu as pltpu

F32 = jnp.float32
BF16 = jnp.bfloat16
EPS = 1e-5
MARGIN = 128
G = 2


def _round_up(a, b):
    return (a + b - 1) // b * b


def _stats_body(x_ref, ssum_ref, ssq_ref, xbf_ref):
    for g in range(G):
        x = x_ref[g]
        ssum_ref[g] = jnp.sum(x, axis=1, keepdims=True)
        ssq_ref[g] = jnp.sum(x * x, axis=1, keepdims=True)
        xbf_ref[g] = x.astype(BF16)


def _bn_coeffs(ssum_ref, ssq_ref, gamma_ref, beta_ref, count):
    s = jnp.sum(ssum_ref[...], axis=0)
    q = jnp.sum(ssq_ref[...], axis=0)
    mean = s / count
    var = jnp.maximum(q / count - mean * mean, 0.0)
    scale = gamma_ref[...] * jax.lax.rsqrt(var + EPS)
    shift = beta_ref[...] - mean * scale
    return scale, shift


def _fill_buf(buf_ref, y, row, w, hw, ch):
    q = jax.lax.broadcasted_iota(jnp.int32, (1, hw), 1) % w
    sl = buf_ref[row:row + 3 * ch, :MARGIN + 2]
    buf_ref[row:row + 3 * ch, :MARGIN + 2] = jnp.zeros_like(sl)
    sr = buf_ref[row:row + 3 * ch, MARGIN + hw - 2:]
    buf_ref[row:row + 3 * ch, MARGIN + hw - 2:] = jnp.zeros_like(sr)
    buf_ref[row + 0 * ch:row + 1 * ch, MARGIN + 1:MARGIN + 1 + hw] = (
        y * (q != w - 1).astype(BF16))
    buf_ref[row + 1 * ch:row + 2 * ch, MARGIN:MARGIN + hw] = y
    buf_ref[row + 2 * ch:row + 3 * ch, MARGIN - 1:MARGIN - 1 + hw] = (
        y * (q != 0).astype(BF16))


def _combine_rows(zbuf_ref, bias, w, hw, cout):
    acc = bias + zbuf_ref[cout:2 * cout, MARGIN:MARGIN + hw]
    acc = acc + zbuf_ref[0:cout, MARGIN - w:MARGIN - w + hw]
    acc = acc + zbuf_ref[2 * cout:3 * cout, MARGIN + w:MARGIN + w + hw]
    return acc


def _zstore(zbuf_ref, z, hw):
    zl = zbuf_ref[:, :MARGIN]
    zbuf_ref[:, :MARGIN] = jnp.zeros_like(zl)
    zr = zbuf_ref[:, MARGIN + hw:]
    zbuf_ref[:, MARGIN + hw:] = jnp.zeros_like(zr)
    zbuf_ref[:, MARGIN:MARGIN + hw] = z


def _conv1_body(x_ref, sx_ref, qx_ref, g1_ref, be1_ref, w_ref, b_ref,
                h1_ref, ssum_ref, ssq_ref, buf_ref, zbuf_ref, *, w, hw, cin,
                cout, count):
    sc, sh = _bn_coeffs(sx_ref, qx_ref, g1_ref, be1_ref, count)
    for g in range(G):
        y = jnp.maximum(x_ref[g].astype(F32) * sc + sh, 0.0).astype(BF16)
        _fill_buf(buf_ref, y, 0, w, hw, cin)
        z = jnp.dot(w_ref[...], buf_ref[:, MARGIN:MARGIN + hw],
                    preferred_element_type=F32)
        _zstore(zbuf_ref, z, hw)
        acc = _combine_rows(zbuf_ref, b_ref[...], w, hw, cout)
        ssum_ref[g] = jnp.sum(acc, axis=1, keepdims=True)
        ssq_ref[g] = jnp.sum(acc * acc, axis=1, keepdims=True)
        h1_ref[g] = acc.astype(BF16)


def _conv23_body(h1_ref, x_ref, sx_ref, qx_ref, s1_ref, q1_ref,
                 gsc_ref, besc_ref, g2_ref, be2_ref, w_ref, b_ref,
                 out_ref, buf_ref, zbuf_ref, *, w, hw, cin, cout, count):
    scs, shs = _bn_coeffs(sx_ref, qx_ref, gsc_ref, besc_ref, count)
    sc2, sh2 = _bn_coeffs(s1_ref, q1_ref, g2_ref, be2_ref, count)
    for g in range(G):
        y2 = jnp.maximum(h1_ref[g].astype(F32) * sc2 + sh2, 0.0).astype(BF16)
        _fill_buf(buf_ref, y2, 0, w, hw, cout)
        y3 = jnp.maximum(x_ref[g].astype(F32) * scs + shs, 0.0).astype(BF16)
        _fill_buf(buf_ref, y3, 3 * cout, w, hw, cin)
        z = jnp.dot(w_ref[...], buf_ref[:, MARGIN:MARGIN + hw],
                    preferred_element_type=F32)
        _zstore(zbuf_ref, z, hw)
        out_ref[g] = _combine_rows(zbuf_ref, b_ref[...], w, hw, cout)


def _wcat(wt):
    co, ci, kh, kw = wt.shape
    return jnp.transpose(wt, (2, 0, 3, 1)).reshape(kh * co, kw * ci)


def kernel(w1, b1, w2, b2, w3, b3, bn1_g, bn1_b, bn2_g, bn2_b,
           bnsc_g, bnsc_b, x):
    n, cin, h, w = x.shape
    cout = w1.shape[0]
    hw = h * w
    lbuf = _round_up(MARGIN + hw + w + 2, 128)
    xr = x.reshape(n, cin, hw)
    steps = n // G
    cnt = float(n * hw)

    sx, qx, xbf = pl.pallas_call(
        _stats_body,
        out_shape=(jax.ShapeDtypeStruct((n, cin, 1), F32),
                   jax.ShapeDtypeStruct((n, cin, 1), F32),
                   jax.ShapeDtypeStruct((n, cin, hw), BF16)),
        grid=(steps,),
        in_specs=[pl.BlockSpec((G, cin, hw), lambda i: (i, 0, 0))],
        out_specs=(pl.BlockSpec((G, cin, 1), lambda i: (i, 0, 0)),
                   pl.BlockSpec((G, cin, 1), lambda i: (i, 0, 0)),
                   pl.BlockSpec((G, cin, hw), lambda i: (i, 0, 0))),
        compiler_params=pltpu.CompilerParams(dimension_semantics=("parallel",)),
    )(xr)

    w1c = _wcat(w1).astype(BF16)
    h1, s1, q1 = pl.pallas_call(
        functools.partial(_conv1_body, w=w, hw=hw, cin=cin, cout=cout,
                          count=cnt),
        out_shape=(jax.ShapeDtypeStruct((n, cout, hw), BF16),
                   jax.ShapeDtypeStruct((n, cout, 1), F32),
                   jax.ShapeDtypeStruct((n, cout, 1), F32)),
        grid=(steps,),
        in_specs=[pl.BlockSpec((G, cin, hw), lambda i: (i, 0, 0)),
                  pl.BlockSpec((n, cin, 1), lambda i: (0, 0, 0)),
                  pl.BlockSpec((n, cin, 1), lambda i: (0, 0, 0)),
                  pl.BlockSpec((cin, 1), lambda i: (0, 0)),
                  pl.BlockSpec((cin, 1), lambda i: (0, 0)),
                  pl.BlockSpec((3 * cout, 3 * cin), lambda i: (0, 0)),
                  pl.BlockSpec((cout, 1), lambda i: (0, 0))],
        out_specs=(pl.BlockSpec((G, cout, hw), lambda i: (i, 0, 0)),
                   pl.BlockSpec((G, cout, 1), lambda i: (i, 0, 0)),
                   pl.BlockSpec((G, cout, 1), lambda i: (i, 0, 0))),
        scratch_shapes=[pltpu.VMEM((3 * cin, lbuf), BF16),
                        pltpu.VMEM((3 * cout, lbuf), F32)],
        compiler_params=pltpu.CompilerParams(
            dimension_semantics=("parallel",),
            vmem_limit_bytes=64 * 1024 * 1024),
    )(xbf, sx, qx, bn1_g[:, None], bn1_b[:, None], w1c, b1[:, None])

    w23c = jnp.concatenate([_wcat(w2), _wcat(w3)], axis=1).astype(BF16)
    b23 = (b2 + b3)[:, None]
    out = pl.pallas_call(
        functools.partial(_conv23_body, w=w, hw=hw, cin=cin, cout=cout,
                          count=cnt),
        out_shape=jax.ShapeDtypeStruct((n, cout, hw), F32),
        grid=(steps,),
        in_specs=[pl.BlockSpec((G, cout, hw), lambda i: (i, 0, 0)),
                  pl.BlockSpec((G, cin, hw), lambda i: (i, 0, 0)),
                  pl.BlockSpec((n, cin, 1), lambda i: (0, 0, 0)),
                  pl.BlockSpec((n, cin, 1), lambda i: (0, 0, 0)),
                  pl.BlockSpec((n, cout, 1), lambda i: (0, 0, 0)),
                  pl.BlockSpec((n, cout, 1), lambda i: (0, 0, 0)),
                  pl.BlockSpec((cin, 1), lambda i: (0, 0)),
                  pl.BlockSpec((cin, 1), lambda i: (0, 0)),
                  pl.BlockSpec((cout, 1), lambda i: (0, 0)),
                  pl.BlockSpec((cout, 1), lambda i: (0, 0)),
                  pl.BlockSpec((3 * cout, 3 * (cin + cout)), lambda i: (0, 0)),
                  pl.BlockSpec((cout, 1), lambda i: (0, 0))],
        out_specs=pl.BlockSpec((G, cout, hw), lambda i: (i, 0, 0)),
        scratch_shapes=[pltpu.VMEM((3 * (cin + cout), lbuf), BF16),
                        pltpu.VMEM((3 * cout, lbuf), F32)],
        compiler_params=pltpu.CompilerParams(
            dimension_semantics=("parallel",),
            vmem_limit_bytes=64 * 1024 * 1024),
    )(h1, xbf, sx, qx, s1, q1, bnsc_g[:, None], bnsc_b[:, None],
      bn2_g[:, None], bn2_b[:, None], w23c, b23)

    return out.reshape(n, cout, h, w)
